# baseline (speedup 1.0000x reference)
import functools

import jax
import jax.numpy as jnp
from jax import lax
from jax.experimental import pallas as pl
from jax.experimental.pallas import tpu as pltpu
from jax.experimental.pallas import tpu_sc as plsc

NC = 2
NS = 16
L = 16
GATHER_CHUNK = 128


def _neumf_sc(B, MF, E, L0, L1):
    NW = NC * NS
    bpw = B // NW
    nblk = bpw // L
    mesh = plsc.VectorSubcoreMesh(core_axis_name="c", subcore_axis_name="s")

    @functools.partial(
        pl.kernel,
        mesh=mesh,
        compiler_params=pltpu.CompilerParams(
            needs_layout_passes=False, use_tc_tiling_on_sc=False),
        out_type=jax.ShapeDtypeStruct((B,), jnp.float32),
        scratch_types=[
            pltpu.VMEM((bpw,), jnp.int32),
            pltpu.VMEM((bpw,), jnp.int32),
            pltpu.VMEM((bpw, MF), jnp.float32),
            pltpu.VMEM((bpw, MF), jnp.float32),
            pltpu.VMEM((bpw, E), jnp.float32),
            pltpu.VMEM((bpw, E), jnp.float32),
            pltpu.VMEM((bpw,), jnp.float32),
            pltpu.VMEM_SHARED((NS, L1, L0), jnp.float32),
            pltpu.VMEM_SHARED((NS, L1), jnp.float32),
            pltpu.VMEM_SHARED((NS, 1, MF + L1), jnp.float32),
            pltpu.VMEM_SHARED((NS, 1), jnp.float32),
            pltpu.SMEM((L1, L0), jnp.float32),
            pltpu.SMEM((L1,), jnp.float32),
            pltpu.SMEM((1, MF + L1), jnp.float32),
            pltpu.SMEM((1,), jnp.float32),
            pltpu.SemaphoreType.DMA,
        ],
    )
    def neumf(uidx_hbm, iidx_hbm, ug_hbm, ig_hbm, um_hbm, im_hbm,
              w1_hbm, b1_hbm, wout_hbm, bout_hbm, out_hbm,
              uidx_v, iidx_v, ug_v, ig_v, um_v, im_v, out_v,
              w1_sh, b1_sh, wout_sh, bout_sh,
              w1_s, b1_s, wout_s, bout_s, dsem):
        sid = lax.axis_index("s")
        wid = sid * NC + lax.axis_index("c")
        base = wid * bpw

        pltpu.sync_copy(uidx_hbm.at[pl.ds(base, bpw)], uidx_v)
        pltpu.sync_copy(iidx_hbm.at[pl.ds(base, bpw)], iidx_v)

        copies = []
        for c in range(bpw // GATHER_CHUNK):
            sl = pl.ds(c * GATHER_CHUNK, GATHER_CHUNK)
            copies.append(pltpu.async_copy(ug_hbm.at[uidx_v.at[sl]], ug_v.at[sl], dsem))
            copies.append(pltpu.async_copy(ig_hbm.at[iidx_v.at[sl]], ig_v.at[sl], dsem))
            copies.append(pltpu.async_copy(um_hbm.at[uidx_v.at[sl]], um_v.at[sl], dsem))
            copies.append(pltpu.async_copy(im_hbm.at[iidx_v.at[sl]], im_v.at[sl], dsem))

        pltpu.sync_copy(w1_hbm, w1_sh.at[sid])
        pltpu.sync_copy(b1_hbm, b1_sh.at[sid])
        pltpu.sync_copy(wout_hbm, wout_sh.at[sid])
        pltpu.sync_copy(bout_hbm, bout_sh.at[sid])
        pltpu.sync_copy(w1_sh.at[sid], w1_s)
        pltpu.sync_copy(b1_sh.at[sid], b1_s)
        pltpu.sync_copy(wout_sh.at[sid], wout_s)
        pltpu.sync_copy(bout_sh.at[sid], bout_s)

        for cp in copies:
            cp.wait()

        lane = lax.iota(jnp.int32, L)
        b1_splat = [jnp.full((L,), b1_s[j], jnp.float32) for j in range(L1)]
        bout_splat = jnp.full((L,), bout_s[0], jnp.float32)

        def block(s, _):
            rows = s * L + lane
            acc = bout_splat
            for d in range(MF):
                dsplat = jnp.full((L,), d, jnp.int32)
                ugT = plsc.load_gather(ug_v, [rows, dsplat])
                igT = plsc.load_gather(ig_v, [rows, dsplat])
                acc = acc + (ugT * igT) * wout_s[0, d]
            hs = list(b1_splat)
            for k in range(L0):
                src = um_v if k < E else im_v
                ksplat = jnp.full((L,), k % E, jnp.int32)
                xk = plsc.load_gather(src, [rows, ksplat])
                for j in range(L1):
                    hs[j] = hs[j] + xk * w1_s[j, k]
            for j in range(L1):
                acc = acc + jnp.maximum(hs[j], 0.0) * wout_s[0, MF + j]
            out_v[pl.ds(s * L, L)] = acc
            return 0

        lax.fori_loop(0, nblk, block, 0)
        pltpu.sync_copy(out_v, out_hbm.at[pl.ds(base, bpw)])

    return neumf


def kernel(user_indices, item_indices, ue_gmf, ie_gmf, ue_mlp, ie_mlp,
           W1, b1, Wout, bout):
    B = user_indices.shape[0]
    MF = ue_gmf.shape[1]
    E = ue_mlp.shape[1]
    L1, L0 = W1.shape
    fn = _neumf_sc(B, MF, E, L0, L1)
    out = fn(user_indices.astype(jnp.int32), item_indices.astype(jnp.int32),
             ue_gmf, ie_gmf, ue_mlp, ie_mlp, W1, b1, Wout, bout)
    return out.reshape(B, 1)
